# baseline (speedup 1.0000x reference)
import functools

import jax
import jax.numpy as jnp
from jax import lax
from jax.experimental import pallas as pl
from jax.experimental.pallas import tpu as pltpu
from jax.experimental.pallas import tpu_sc as plsc

_D = 64
VOCAB_ROWS = 1000000
_SCALE = float(_D) ** 0.5
_NC, _NS = 2, 16
_NW = _NC * _NS
_ROWS = 4096
_CH = 200
_RPW = _ROWS // _NW
_NBUF = 4

_mesh = plsc.VectorSubcoreMesh(core_axis_name="c", subcore_axis_name="s")


@functools.partial(
    pl.kernel,
    out_type=jax.ShapeDtypeStruct((_ROWS, _CH, 2 * _D), jnp.float32),
    mesh=_mesh,
    compiler_params=pltpu.CompilerParams(use_tc_tiling_on_sc=False),
    scratch_types=[
        pltpu.VMEM((_RPW, _CH), jnp.int32),
        pltpu.VMEM((_RPW, _CH), jnp.int32),
        [pltpu.VMEM((_CH, _D), jnp.float32) for _ in range(_NBUF)],
        [pltpu.SemaphoreType.DMA for _ in range(_NBUF)],
        [pltpu.SemaphoreType.DMA for _ in range(_NBUF)],
    ],
)
def _emb_lookup(table, idx, out, idx_v, idx2_v, bufs, gsems, wsems):
    wid = lax.axis_index("s") * _NC + lax.axis_index("c")
    row_base = wid * _RPW
    pltpu.sync_copy(idx.at[pl.ds(row_base, _RPW)], idx_v)

    @pl.loop(0, _RPW)
    def _dbl(r):
        for c in range(_CH // 16 + 1):
            sl = pl.ds(min(c * 16, _CH - 16), 16)
            idx2_v[r, sl] = idx_v[r, sl] * 2

    def start_gather(j, b):
        pltpu.async_copy(table.at[idx2_v.at[j]], bufs[b], gsems[b])

    def wait_gather(j, b):
        pltpu.make_async_copy(table.at[idx2_v.at[j]], bufs[b], gsems[b]).wait()

    def start_wb(j, b):
        pltpu.async_copy(bufs[b], out.at[row_base + j, :, pl.ds(0, _D)], wsems[b])

    def wait_wb(j, b):
        pltpu.make_async_copy(bufs[b], out.at[row_base + j, :, pl.ds(0, _D)], wsems[b]).wait()

    def scale(b):
        buf = bufs[b]

        @pl.loop(0, _CH, unroll=8)
        def _row(r):
            for c in range(_D // 16):
                sl = pl.ds(c * 16, 16)
                buf[r, sl] = buf[r, sl] * _SCALE

    def step(j, b, first=False, tail=False):
        wait_gather(j, b)
        scale(b)
        start_wb(j, b)
        if not first:
            wait_wb(j - 1, (b - 1) % _NBUF)
        if not tail:
            start_gather(j + _NBUF - 1, (b - 1) % _NBUF)

    for b in range(_NBUF - 1):
        start_gather(b, b)

    for b in range(_NBUF):
        step(b, b, first=(b == 0))

    @pl.loop(_NBUF, _RPW - _NBUF, step=_NBUF)
    def _block(j0):
        for b in range(_NBUF):
            step(j0 + b, b)

    for b in range(_NBUF):
        j = _RPW - _NBUF + b
        step(j, b, tail=(j + _NBUF - 1 >= _RPW))

    wait_wb(_RPW - 1, (_RPW - 1) % _NBUF)


def kernel(inputs, embeddings):
    t128 = jnp.concatenate(
        [embeddings, jnp.zeros((VOCAB_ROWS, _D), jnp.float32)], axis=1
    )
    t2m = jnp.reshape(t128, (2 * VOCAB_ROWS, _D))
    out128 = _emb_lookup(t2m, inputs)
    return out128[:, :, :_D]


# speedup vs baseline: 1.2732x; 1.2732x over previous
import functools

import jax
import jax.numpy as jnp
from jax import lax
from jax.experimental import pallas as pl
from jax.experimental.pallas import tpu as pltpu
from jax.experimental.pallas import tpu_sc as plsc

_D = 64
VOCAB_ROWS = 1000000
_SCALE = float(_D) ** 0.5
_NC, _NS = 2, 16
_NW = _NC * _NS
_ROWS = 4096
_CH = 200
_RPW = _ROWS // _NW
_NBUF = 4

_mesh = plsc.VectorSubcoreMesh(core_axis_name="c", subcore_axis_name="s")


@functools.partial(
    pl.kernel,
    out_type=jax.ShapeDtypeStruct((_ROWS, _CH, 2 * _D), jnp.float32),
    mesh=_mesh,
    compiler_params=pltpu.CompilerParams(use_tc_tiling_on_sc=False),
    scratch_types=[
        pltpu.VMEM((_RPW, _CH), jnp.int32),
        pltpu.VMEM((_RPW, _CH), jnp.int32),
        [pltpu.VMEM((_CH, _D), jnp.float32) for _ in range(_NBUF)],
        [pltpu.SemaphoreType.DMA for _ in range(_NBUF)],
        [pltpu.SemaphoreType.DMA for _ in range(_NBUF)],
    ],
)
def _emb_lookup(table, idx, out, idx_v, idx2_v, bufs, gsems, wsems):
    wid = lax.axis_index("s") * _NC + lax.axis_index("c")
    row_base = wid * _RPW
    pltpu.sync_copy(idx.at[pl.ds(row_base, _RPW)], idx_v)

    @pl.loop(0, _RPW)
    def _dbl(r):
        for c in range(_CH // 16 + 1):
            sl = pl.ds(min(c * 16, _CH - 16), 16)
            idx2_v[r, sl] = idx_v[r, sl] * 2

    def start_gather(j, b):
        pltpu.async_copy(table.at[idx2_v.at[j]], bufs[b], gsems[b])

    def wait_gather(j, b):
        pltpu.make_async_copy(table.at[idx2_v.at[j]], bufs[b], gsems[b]).wait()

    def start_wb(j, b):
        pltpu.async_copy(bufs[b], out.at[row_base + j, :, pl.ds(0, _D)], wsems[b])

    def wait_wb(j, b):
        pltpu.make_async_copy(bufs[b], out.at[row_base + j, :, pl.ds(0, _D)], wsems[b]).wait()

    def scale(b):
        buf = bufs[b]

        @pl.loop(0, _CH, unroll=8)
        def _row(r):
            for c in range(_D // 16):
                sl = pl.ds(c * 16, 16)
                buf[r, sl] = buf[r, sl] * _SCALE

    def step(j, b, first=False, tail=False):
        wait_gather(j, b)
        scale(b)
        start_wb(j, b)
        if not first:
            wait_wb(j - 1, (b - 1) % _NBUF)
        if not tail:
            start_gather(j + _NBUF - 1, (b - 1) % _NBUF)

    for b in range(_NBUF - 1):
        start_gather(b, b)

    for b in range(_NBUF):
        step(b, b, first=(b == 0))

    @pl.loop(_NBUF, _RPW - _NBUF, step=_NBUF)
    def _block(j0):
        for b in range(_NBUF):
            step(j0 + b, b)

    for b in range(_NBUF):
        j = _RPW - _NBUF + b
        step(j, b, tail=(j + _NBUF - 1 >= _RPW))

    wait_wb(_RPW - 1, (_RPW - 1) % _NBUF)


_TBLK = 4096


def _tc_transpose_body(src_ref, dst_ref):
    dst_ref[:, :_D] = src_ref[...].T


_tc_transpose = pl.pallas_call(
    _tc_transpose_body,
    grid=((VOCAB_ROWS + _TBLK - 1) // _TBLK,),
    in_specs=[pl.BlockSpec((_D, _TBLK), lambda i: (0, i))],
    out_specs=pl.BlockSpec((_TBLK, 2 * _D), lambda i: (i, 0)),
    out_shape=jax.ShapeDtypeStruct((VOCAB_ROWS, 2 * _D), jnp.float32),
)


def kernel(inputs, embeddings):
    t128 = _tc_transpose(embeddings.T)
    t2m = jnp.reshape(t128, (2 * VOCAB_ROWS, _D))
    out128 = _emb_lookup(t2m, inputs)
    return out128[:, :, :_D]


# speedup vs baseline: 1.2975x; 1.0191x over previous
import functools

import jax
import jax.numpy as jnp
from jax import lax
from jax.experimental import pallas as pl
from jax.experimental.pallas import tpu as pltpu
from jax.experimental.pallas import tpu_sc as plsc

_D = 64
VOCAB_ROWS = 1000000
_SCALE = float(_D) ** 0.5
_NC, _NS = 2, 16
_NW = _NC * _NS
_ROWS = 4096
_CH = 200
_RPW = _ROWS // _NW
_NBUF = 4
_TBLK = 4096
_HPAIR = 245 * (_TBLK // 2)

_mesh = plsc.VectorSubcoreMesh(core_axis_name="c", subcore_axis_name="s")


@functools.partial(
    pl.kernel,
    out_type=jax.ShapeDtypeStruct((_ROWS, _CH, 2 * _D), jnp.float32),
    mesh=_mesh,
    compiler_params=pltpu.CompilerParams(use_tc_tiling_on_sc=False),
    scratch_types=[
        pltpu.VMEM((_RPW, _CH), jnp.int32),
        pltpu.VMEM((_RPW, _CH), jnp.int32),
        [pltpu.VMEM((_CH, _D), jnp.float32) for _ in range(_NBUF)],
        [pltpu.SemaphoreType.DMA for _ in range(_NBUF)],
        [pltpu.SemaphoreType.DMA for _ in range(_NBUF)],
    ],
)
def _emb_lookup(table, idx, out, idx_v, idx2_v, bufs, gsems, wsems):
    wid = lax.axis_index("s") * _NC + lax.axis_index("c")
    row_base = wid * _RPW
    pltpu.sync_copy(idx.at[pl.ds(row_base, _RPW)], idx_v)

    @pl.loop(0, _RPW)
    def _remap(r):
        for c in range(_CH // 16 + 1):
            sl = pl.ds(min(c * 16, _CH - 16), 16)
            v = idx_v[r, sl]
            idx2_v[r, sl] = v * 2 - jnp.where(v >= _HPAIR, 2 * _HPAIR - 1, 0)

    def start_gather(j, b):
        pltpu.async_copy(table.at[idx2_v.at[j]], bufs[b], gsems[b])

    def wait_gather(j, b):
        pltpu.make_async_copy(table.at[idx2_v.at[j]], bufs[b], gsems[b]).wait()

    def start_wb(j, b):
        pltpu.async_copy(bufs[b], out.at[row_base + j, :, pl.ds(0, _D)], wsems[b])

    def wait_wb(j, b):
        pltpu.make_async_copy(bufs[b], out.at[row_base + j, :, pl.ds(0, _D)], wsems[b]).wait()

    def step(j, b, first=False, tail=False):
        wait_gather(j, b)
        start_wb(j, b)
        if not first:
            wait_wb(j - 1, (b - 1) % _NBUF)
        if not tail:
            start_gather(j + _NBUF - 1, (b - 1) % _NBUF)

    for b in range(_NBUF - 1):
        start_gather(b, b)

    for b in range(_NBUF):
        step(b, b, first=(b == 0))

    @pl.loop(_NBUF, _RPW - _NBUF, step=_NBUF)
    def _block(j0):
        for b in range(_NBUF):
            step(j0 + b, b)

    for b in range(_NBUF):
        j = _RPW - _NBUF + b
        step(j, b, tail=(j + _NBUF - 1 >= _RPW))

    wait_wb(_RPW - 1, (_RPW - 1) % _NBUF)


_HB = _TBLK // 2
_NHB = _HPAIR // _HB
_NEDGE = VOCAB_ROWS // _HB


def _tc_pack_body(lo_ref, hi_ref, dst_ref):
    dst_ref[:, :_D] = (lo_ref[...] * _SCALE).T
    dst_ref[:, _D:] = (hi_ref[...] * _SCALE).T


_tc_pack = pl.pallas_call(
    _tc_pack_body,
    grid=(_NHB,),
    in_specs=[
        pl.BlockSpec((_D, _HB), lambda i: (0, i)),
        pl.BlockSpec((_D, _HB), lambda i: (0, jnp.minimum(i + _NHB, _NEDGE))),
    ],
    out_specs=pl.BlockSpec((_HB, 2 * _D), lambda i: (i, 0)),
    out_shape=jax.ShapeDtypeStruct((_HPAIR, 2 * _D), jnp.float32),
)


def kernel(inputs, embeddings):
    tp = _tc_pack(embeddings.T, embeddings.T)
    tflat = jnp.reshape(tp, (2 * _HPAIR, _D))
    out128 = _emb_lookup(tflat, inputs)
    return out128[:, :, :_D]


# speedup vs baseline: 1.4462x; 1.1146x over previous
import functools

import jax
import jax.numpy as jnp
from jax import lax
from jax.experimental import pallas as pl
from jax.experimental.pallas import tpu as pltpu
from jax.experimental.pallas import tpu_sc as plsc

_D = 64
VOCAB_ROWS = 1000000
_SCALE = float(_D) ** 0.5
_NC, _NS = 2, 16
_NW = _NC * _NS
_ROWS = 4096
_CH = 200
_RPW = _ROWS // _NW
_NBUF = 4
_TBLK = 8192
_HPAIR = 123 * (_TBLK // 2)

_mesh = plsc.VectorSubcoreMesh(core_axis_name="c", subcore_axis_name="s")


@functools.partial(
    pl.kernel,
    out_type=jax.ShapeDtypeStruct((_ROWS, _CH, 2 * _D), jnp.float32),
    mesh=_mesh,
    compiler_params=pltpu.CompilerParams(use_tc_tiling_on_sc=False),
    scratch_types=[
        pltpu.VMEM((_RPW, _CH), jnp.int32),
        pltpu.VMEM((_RPW, _CH), jnp.int32),
        [pltpu.VMEM((_CH, _D), jnp.float32) for _ in range(_NBUF)],
        [pltpu.SemaphoreType.DMA for _ in range(_NBUF)],
        [pltpu.SemaphoreType.DMA for _ in range(_NBUF)],
    ],
)
def _emb_lookup(table, idx, out, idx_v, idx2_v, bufs, gsems, wsems):
    wid = lax.axis_index("s") * _NC + lax.axis_index("c")
    row_base = wid * _RPW
    pltpu.sync_copy(idx.at[pl.ds(row_base, _RPW)], idx_v)

    @pl.loop(0, _RPW)
    def _remap(r):
        for c in range(_CH // 16 + 1):
            sl = pl.ds(min(c * 16, _CH - 16), 16)
            v = idx_v[r, sl]
            idx2_v[r, sl] = v * 2 - jnp.where(v >= _HPAIR, 2 * _HPAIR - 1, 0)

    def start_gather(j, b):
        pltpu.async_copy(table.at[idx2_v.at[j]], bufs[b], gsems[b])

    def wait_gather(j, b):
        pltpu.make_async_copy(table.at[idx2_v.at[j]], bufs[b], gsems[b]).wait()

    def start_wb(j, b):
        pltpu.async_copy(bufs[b], out.at[row_base + j, :, pl.ds(0, _D)], wsems[b])

    def wait_wb(j, b):
        pltpu.make_async_copy(bufs[b], out.at[row_base + j, :, pl.ds(0, _D)], wsems[b]).wait()

    def step(j, b, first=False, tail=False):
        wait_gather(j, b)
        start_wb(j, b)
        if not first:
            wait_wb(j - 1, (b - 1) % _NBUF)
        if not tail:
            start_gather(j + _NBUF - 1, (b - 1) % _NBUF)

    for b in range(_NBUF - 1):
        start_gather(b, b)

    for b in range(_NBUF):
        step(b, b, first=(b == 0))

    @pl.loop(_NBUF, _RPW - _NBUF, step=_NBUF)
    def _block(j0):
        for b in range(_NBUF):
            step(j0 + b, b)

    for b in range(_NBUF):
        j = _RPW - _NBUF + b
        step(j, b, tail=(j + _NBUF - 1 >= _RPW))

    wait_wb(_RPW - 1, (_RPW - 1) % _NBUF)


_HB = _TBLK // 2
_NHB = _HPAIR // _HB
_NEDGE = VOCAB_ROWS // _HB


def _tc_pack_body(lo_ref, hi_ref, dst_ref):
    dst_ref[:, :_D] = (lo_ref[...] * _SCALE).T
    dst_ref[:, _D:] = (hi_ref[...] * _SCALE).T


_tc_pack = pl.pallas_call(
    _tc_pack_body,
    grid=(_NHB,),
    in_specs=[
        pl.BlockSpec((_D, _HB), lambda i: (0, i)),
        pl.BlockSpec((_D, _HB), lambda i: (0, jnp.minimum(i + _NHB, _NEDGE))),
    ],
    out_specs=pl.BlockSpec((_HB, 2 * _D), lambda i: (i, 0)),
    out_shape=jax.ShapeDtypeStruct((_HPAIR, 2 * _D), jnp.float32),
)


def kernel(inputs, embeddings):
    tp = _tc_pack(embeddings.T, embeddings.T)
    tflat = jnp.reshape(tp, (2 * _HPAIR, _D))
    out128 = _emb_lookup(tflat, inputs)
    return out128[:, :, :_D]


# speedup vs baseline: 1.5273x; 1.0560x over previous
import functools

import jax
import jax.numpy as jnp
from jax import lax
from jax.experimental import pallas as pl
from jax.experimental.pallas import tpu as pltpu
from jax.experimental.pallas import tpu_sc as plsc

_D = 64
VOCAB_ROWS = 1000000
_SCALE = float(_D) ** 0.5
_NC, _NS = 2, 16
_NW = _NC * _NS
_ROWS = 4096
_CH = 200
_RPW = _ROWS // _NW
_NBUF = 4
_TBLK = 16384
_HPAIR = 62 * (_TBLK // 2)

_mesh = plsc.VectorSubcoreMesh(core_axis_name="c", subcore_axis_name="s")


@functools.partial(
    pl.kernel,
    out_type=jax.ShapeDtypeStruct((_ROWS, _CH, 2 * _D), jnp.float32),
    mesh=_mesh,
    compiler_params=pltpu.CompilerParams(use_tc_tiling_on_sc=False),
    scratch_types=[
        pltpu.VMEM((_RPW, _CH), jnp.int32),
        pltpu.VMEM((_RPW, _CH), jnp.int32),
        [pltpu.VMEM((_CH, _D), jnp.float32) for _ in range(_NBUF)],
        [pltpu.SemaphoreType.DMA for _ in range(_NBUF)],
        [pltpu.SemaphoreType.DMA for _ in range(_NBUF)],
    ],
)
def _emb_lookup(table, idx, out, idx_v, idx2_v, bufs, gsems, wsems):
    wid = lax.axis_index("s") * _NC + lax.axis_index("c")
    row_base = wid * _RPW
    pltpu.sync_copy(idx.at[pl.ds(row_base, _RPW)], idx_v)

    @pl.loop(0, _RPW)
    def _remap(r):
        for c in range(_CH // 16 + 1):
            sl = pl.ds(min(c * 16, _CH - 16), 16)
            v = idx_v[r, sl]
            idx2_v[r, sl] = v * 2 - jnp.where(v >= _HPAIR, 2 * _HPAIR - 1, 0)

    def start_gather(j, b):
        pltpu.async_copy(table.at[idx2_v.at[j]], bufs[b], gsems[b])

    def wait_gather(j, b):
        pltpu.make_async_copy(table.at[idx2_v.at[j]], bufs[b], gsems[b]).wait()

    def start_wb(j, b):
        pltpu.async_copy(bufs[b], out.at[row_base + j, :, pl.ds(0, _D)], wsems[b])

    def wait_wb(j, b):
        pltpu.make_async_copy(bufs[b], out.at[row_base + j, :, pl.ds(0, _D)], wsems[b]).wait()

    def step(j, b, first=False, tail=False):
        wait_gather(j, b)
        start_wb(j, b)
        if not first:
            wait_wb(j - 1, (b - 1) % _NBUF)
        if not tail:
            start_gather(j + _NBUF - 1, (b - 1) % _NBUF)

    for b in range(_NBUF - 1):
        start_gather(b, b)

    for b in range(_NBUF):
        step(b, b, first=(b == 0))

    @pl.loop(_NBUF, _RPW - _NBUF, step=_NBUF)
    def _block(j0):
        for b in range(_NBUF):
            step(j0 + b, b)

    for b in range(_NBUF):
        j = _RPW - _NBUF + b
        step(j, b, tail=(j + _NBUF - 1 >= _RPW))

    wait_wb(_RPW - 1, (_RPW - 1) % _NBUF)


_HB = _TBLK // 2
_NHB = _HPAIR // _HB
_NEDGE = VOCAB_ROWS // _HB


def _tc_pack_body(lo_ref, hi_ref, dst_ref):
    dst_ref[:, :_D] = (lo_ref[...] * _SCALE).T
    dst_ref[:, _D:] = (hi_ref[...] * _SCALE).T


_tc_pack = pl.pallas_call(
    _tc_pack_body,
    grid=(_NHB,),
    in_specs=[
        pl.BlockSpec((_D, _HB), lambda i: (0, i)),
        pl.BlockSpec((_D, _HB), lambda i: (0, jnp.minimum(i + _NHB, _NEDGE))),
    ],
    out_specs=pl.BlockSpec((_HB, 2 * _D), lambda i: (i, 0)),
    out_shape=jax.ShapeDtypeStruct((_HPAIR, 2 * _D), jnp.float32),
)


def kernel(inputs, embeddings):
    tp = _tc_pack(embeddings.T, embeddings.T)
    tflat = jnp.reshape(tp, (2 * _HPAIR, _D))
    out128 = _emb_lookup(tflat, inputs)
    return out128[:, :, :_D]


# speedup vs baseline: 1.5641x; 1.0241x over previous
import functools

import jax
import jax.numpy as jnp
from jax import lax
from jax.experimental import pallas as pl
from jax.experimental.pallas import tpu as pltpu
from jax.experimental.pallas import tpu_sc as plsc

_D = 64
VOCAB_ROWS = 1000000
_SCALE = float(_D) ** 0.5
_NC, _NS = 2, 16
_NW = _NC * _NS
_ROWS = 4096
_CH = 200
_RPW = _ROWS // _NW
_NBUF = 4
_TBLK = 32768
_HPAIR = 31 * (_TBLK // 2)

_mesh = plsc.VectorSubcoreMesh(core_axis_name="c", subcore_axis_name="s")


@functools.partial(
    pl.kernel,
    out_type=jax.ShapeDtypeStruct((_ROWS, _CH, 2 * _D), jnp.float32),
    mesh=_mesh,
    compiler_params=pltpu.CompilerParams(use_tc_tiling_on_sc=False),
    scratch_types=[
        pltpu.VMEM((_RPW, _CH), jnp.int32),
        pltpu.VMEM((_RPW, _CH), jnp.int32),
        [pltpu.VMEM((_CH, _D), jnp.float32) for _ in range(_NBUF)],
        [pltpu.SemaphoreType.DMA for _ in range(_NBUF)],
        [pltpu.SemaphoreType.DMA for _ in range(_NBUF)],
    ],
)
def _emb_lookup(table, idx, out, idx_v, idx2_v, bufs, gsems, wsems):
    wid = lax.axis_index("s") * _NC + lax.axis_index("c")
    row_base = wid * _RPW
    pltpu.sync_copy(idx.at[pl.ds(row_base, _RPW)], idx_v)

    @pl.loop(0, _RPW)
    def _remap(r):
        for c in range(_CH // 16 + 1):
            sl = pl.ds(min(c * 16, _CH - 16), 16)
            v = idx_v[r, sl]
            idx2_v[r, sl] = v * 2 - jnp.where(v >= _HPAIR, 2 * _HPAIR - 1, 0)

    def start_gather(j, b):
        pltpu.async_copy(table.at[idx2_v.at[j]], bufs[b], gsems[b])

    def wait_gather(j, b):
        pltpu.make_async_copy(table.at[idx2_v.at[j]], bufs[b], gsems[b]).wait()

    def start_wb(j, b):
        pltpu.async_copy(bufs[b], out.at[row_base + j, :, pl.ds(0, _D)], wsems[b])

    def wait_wb(j, b):
        pltpu.make_async_copy(bufs[b], out.at[row_base + j, :, pl.ds(0, _D)], wsems[b]).wait()

    def step(j, b, first=False, tail=False):
        wait_gather(j, b)
        start_wb(j, b)
        if not first:
            wait_wb(j - 1, (b - 1) % _NBUF)
        if not tail:
            start_gather(j + _NBUF - 1, (b - 1) % _NBUF)

    for b in range(_NBUF - 1):
        start_gather(b, b)

    for b in range(_NBUF):
        step(b, b, first=(b == 0))

    @pl.loop(_NBUF, _RPW - _NBUF, step=_NBUF)
    def _block(j0):
        for b in range(_NBUF):
            step(j0 + b, b)

    for b in range(_NBUF):
        j = _RPW - _NBUF + b
        step(j, b, tail=(j + _NBUF - 1 >= _RPW))

    wait_wb(_RPW - 1, (_RPW - 1) % _NBUF)


_HB = _TBLK // 2
_NHB = _HPAIR // _HB
_NEDGE = VOCAB_ROWS // _HB


def _tc_pack_body(lo_ref, hi_ref, dst_ref):
    dst_ref[:, :_D] = (lo_ref[...] * _SCALE).T
    dst_ref[:, _D:] = (hi_ref[...] * _SCALE).T


_tc_pack = pl.pallas_call(
    _tc_pack_body,
    grid=(_NHB,),
    in_specs=[
        pl.BlockSpec((_D, _HB), lambda i: (0, i)),
        pl.BlockSpec((_D, _HB), lambda i: (0, jnp.minimum(i + _NHB, _NEDGE))),
    ],
    out_specs=pl.BlockSpec((_HB, 2 * _D), lambda i: (i, 0)),
    out_shape=jax.ShapeDtypeStruct((_HPAIR, 2 * _D), jnp.float32),
)


def kernel(inputs, embeddings):
    tp = _tc_pack(embeddings.T, embeddings.T)
    tflat = jnp.reshape(tp, (2 * _HPAIR, _D))
    out128 = _emb_lookup(tflat, inputs)
    return out128[:, :, :_D]
